# initial kernel scaffold (unmeasured)
import jax
import jax.numpy as jnp
from jax import lax
from jax.experimental import pallas as pl
from jax.experimental.pallas import tpu as pltpu

N_DEV = 32
N_EXP_LOCAL = 4
N_EXP = N_DEV * N_EXP_LOCAL


def kernel(x, router_W, route_idx, expert_W):
    n_tok, d_model = x.shape
    d_ff = expert_W.shape[-1]
    n_exp = router_W.shape[-1]

    ew_bf = expert_W.astype(jnp.bfloat16)

    def body(x_ref, rw_ref, idx_ref, ew_ref, out_ref, w_all, send_sems, recv_sems):
        me = lax.axis_index("i")
        left = lax.rem(me + N_DEV - 1, N_DEV)
        right = lax.rem(me + 1, N_DEV)

        barrier_sem = pltpu.get_barrier_semaphore()
        for nbr in (left, right):
            pl.semaphore_signal(
                barrier_sem, inc=1,
                device_id=(nbr,), device_id_type=pl.DeviceIdType.MESH,
            )
        pl.semaphore_wait(barrier_sem, 2)

        w_all[pl.ds(me * N_EXP_LOCAL, N_EXP_LOCAL)] = ew_ref[...]

        for h in range(N_DEV - 1):
            o = lax.rem(me - h + N_DEV, N_DEV)
            rdma = pltpu.make_async_remote_copy(
                src_ref=w_all.at[pl.ds(o * N_EXP_LOCAL, N_EXP_LOCAL)],
                dst_ref=w_all.at[pl.ds(o * N_EXP_LOCAL, N_EXP_LOCAL)],
                send_sem=send_sems.at[h],
                recv_sem=recv_sems.at[h],
                device_id=(right,),
                device_id_type=pl.DeviceIdType.MESH,
            )
            rdma.start()
            rdma.wait()

        xf = x_ref[...]
        scores = jnp.dot(xf, rw_ref[...], preferred_element_type=jnp.float32)
        s_max = jnp.max(scores, axis=-1, keepdims=True)
        p = jnp.exp(scores - s_max)
        probs = p / jnp.sum(p, axis=-1, keepdims=True)

        ids = lax.broadcasted_iota(jnp.int32, (n_tok, n_exp), 1)
        oh0 = ids == idx_ref[:, 0:1]
        oh1 = ids == idx_ref[:, 1:2]
        g0 = jnp.sum(jnp.where(oh0, probs, 0.0), axis=-1, keepdims=True)
        g1 = jnp.sum(jnp.where(oh1, probs, 0.0), axis=-1, keepdims=True)
        gate = probs * (oh0 | oh1).astype(jnp.float32) / (g0 + g1)

        xb = xf.astype(jnp.bfloat16)
        out_ref[...] = jnp.zeros((n_tok, d_ff), jnp.float32)

        def eloop(e, _):
            w = w_all[e]
            d = jnp.dot(xb, w, preferred_element_type=jnp.float32)
            g = lax.dynamic_slice(gate, (0, e), (n_tok, 1))
            out_ref[...] = out_ref[...] + g * d
            return 0

        lax.fori_loop(0, n_exp, eloop, 0)

    return pl.pallas_call(
        body,
        out_shape=jax.ShapeDtypeStruct((n_tok, d_ff), jnp.float32),
        in_specs=[
            pl.BlockSpec(memory_space=pltpu.VMEM),
            pl.BlockSpec(memory_space=pltpu.VMEM),
            pl.BlockSpec(memory_space=pltpu.VMEM),
            pl.BlockSpec(memory_space=pltpu.VMEM),
        ],
        out_specs=pl.BlockSpec(memory_space=pltpu.VMEM),
        scratch_shapes=[
            pltpu.VMEM((N_EXP, 256, 512), jnp.bfloat16),
            pltpu.SemaphoreType.DMA((N_DEV - 1,)),
            pltpu.SemaphoreType.DMA((N_DEV - 1,)),
        ],
        compiler_params=pltpu.CompilerParams(collective_id=0),
    )(x, router_W, route_idx, ew_bf)


# baseline (device time: 438097 ns/iter reference)
import jax
import jax.numpy as jnp
from jax import lax
from jax.experimental import pallas as pl
from jax.experimental.pallas import tpu as pltpu

N_DEV = 32
N_EXP_LOCAL = 4
N_EXP = N_DEV * N_EXP_LOCAL


def kernel(x, router_W, route_idx, expert_W):
    n_tok, d_model = x.shape
    d_ff = expert_W.shape[-1]
    n_exp = router_W.shape[-1]

    ew_bf = expert_W.astype(jnp.bfloat16)

    def body(x_ref, rw_ref, idx_ref, ew_ref, out_ref, w_all, send_sems, recv_sems):
        me = lax.axis_index("i")
        left = lax.rem(me + N_DEV - 1, N_DEV)
        right = lax.rem(me + 1, N_DEV)

        barrier_sem = pltpu.get_barrier_semaphore()
        for nbr in (left, right):
            pl.semaphore_signal(
                barrier_sem, inc=1,
                device_id=(nbr,), device_id_type=pl.DeviceIdType.MESH,
            )
        pl.semaphore_wait(barrier_sem, 2)

        w_all[me] = jnp.reshape(ew_ref[...], (N_EXP_LOCAL * 256, 512))

        for h in range(N_DEV - 1):
            o = lax.rem(me - h + N_DEV, N_DEV)
            rdma = pltpu.make_async_remote_copy(
                src_ref=w_all.at[o],
                dst_ref=w_all.at[o],
                send_sem=send_sems.at[h],
                recv_sem=recv_sems.at[h],
                device_id=(right,),
                device_id_type=pl.DeviceIdType.MESH,
            )
            rdma.start()
            rdma.wait()

        xf = x_ref[...]
        scores = jnp.dot(xf, rw_ref[...], preferred_element_type=jnp.float32)
        s_max = jnp.max(scores, axis=-1, keepdims=True)
        p = jnp.exp(scores - s_max)
        probs = p / jnp.sum(p, axis=-1, keepdims=True)

        ids = lax.broadcasted_iota(jnp.int32, (n_tok, n_exp), 1)
        oh0 = ids == idx_ref[:, 0:1]
        oh1 = ids == idx_ref[:, 1:2]
        g0 = jnp.sum(jnp.where(oh0, probs, 0.0), axis=-1, keepdims=True)
        g1 = jnp.sum(jnp.where(oh1, probs, 0.0), axis=-1, keepdims=True)
        gate = probs * (oh0 | oh1).astype(jnp.float32) / (g0 + g1)

        xb = xf.astype(jnp.bfloat16)
        gate_bf = gate.astype(jnp.bfloat16)
        out_ref[...] = jnp.zeros((n_tok, d_ff), jnp.float32)
        for c in range(N_DEV):
            xg = jnp.concatenate(
                [
                    xb * gate_bf[:, N_EXP_LOCAL * c + j : N_EXP_LOCAL * c + j + 1]
                    for j in range(N_EXP_LOCAL)
                ],
                axis=1,
            )
            out_ref[...] = out_ref[...] + jnp.dot(
                xg, w_all[c], preferred_element_type=jnp.float32
            )

    return pl.pallas_call(
        body,
        out_shape=jax.ShapeDtypeStruct((n_tok, d_ff), jnp.float32),
        in_specs=[
            pl.BlockSpec(memory_space=pltpu.VMEM),
            pl.BlockSpec(memory_space=pltpu.VMEM),
            pl.BlockSpec(memory_space=pltpu.VMEM),
            pl.BlockSpec(memory_space=pltpu.VMEM),
        ],
        out_specs=pl.BlockSpec(memory_space=pltpu.VMEM),
        scratch_shapes=[
            pltpu.VMEM((N_DEV, N_EXP_LOCAL * 256, 512), jnp.bfloat16),
            pltpu.SemaphoreType.DMA((N_DEV - 1,)),
            pltpu.SemaphoreType.DMA((N_DEV - 1,)),
        ],
        compiler_params=pltpu.CompilerParams(
            collective_id=0,
            vmem_limit_bytes=100 * 1024 * 1024,
        ),
    )(x, router_W, route_idx, ew_bf)


# device time: 413714 ns/iter; 1.0589x vs baseline; 1.0589x over previous
import jax
import jax.numpy as jnp
from jax import lax
from jax.experimental import pallas as pl
from jax.experimental.pallas import tpu as pltpu

N_DEV = 32
N_EXP_LOCAL = 4
HALF = N_EXP_LOCAL // 2


def kernel(x, router_W, route_idx, expert_W):
    n_tok, d_model = x.shape
    d_ff = expert_W.shape[-1]
    n_exp = router_W.shape[-1]
    k_half = HALF * d_model

    ew_bf = expert_W.astype(jnp.bfloat16)

    def body(x_ref, rw_ref, idx_ref, ew_ref, out_ref,
             w_r, w_l, gate_s,
             send_r, recv_r, send_l, recv_l):
        me = lax.axis_index("i")
        left = lax.rem(me + N_DEV - 1, N_DEV)
        right = lax.rem(me + 1, N_DEV)

        barrier_sem = pltpu.get_barrier_semaphore()
        for nbr in (left, right):
            pl.semaphore_signal(
                barrier_sem, inc=1,
                device_id=(nbr,), device_id_type=pl.DeviceIdType.MESH,
            )
        pl.semaphore_wait(barrier_sem, 2)

        w_r[me] = jnp.reshape(ew_ref[0:HALF], (k_half, d_ff))
        w_l[me] = jnp.reshape(ew_ref[HALF:N_EXP_LOCAL], (k_half, d_ff))

        xf = x_ref[...]
        scores = jnp.dot(xf, rw_ref[...], preferred_element_type=jnp.float32)
        s_max = jnp.max(scores, axis=-1, keepdims=True)
        p = jnp.exp(scores - s_max)
        probs = p / jnp.sum(p, axis=-1, keepdims=True)
        ids = lax.broadcasted_iota(jnp.int32, (n_tok, n_exp), 1)
        oh0 = ids == idx_ref[:, 0:1]
        oh1 = ids == idx_ref[:, 1:2]
        g0 = jnp.sum(jnp.where(oh0, probs, 0.0), axis=-1, keepdims=True)
        g1 = jnp.sum(jnp.where(oh1, probs, 0.0), axis=-1, keepdims=True)
        gate = probs * (oh0 | oh1).astype(jnp.float32) / (g0 + g1)

        for o in range(N_DEV):
            gate_s[o] = gate[:, N_EXP_LOCAL * o : N_EXP_LOCAL * (o + 1)]

        xb = xf.astype(jnp.bfloat16)
        out_ref[...] = jnp.zeros((n_tok, d_ff), jnp.float32)

        def consume(o, j0, w_ref_slot):
            g4 = gate_s[o]
            xg = jnp.concatenate(
                [
                    xb * g4[:, j0 + j : j0 + j + 1].astype(jnp.bfloat16)
                    for j in range(HALF)
                ],
                axis=1,
            )
            out_ref[...] = out_ref[...] + jnp.dot(
                xg, w_ref_slot, preferred_element_type=jnp.float32
            )

        for h in range(N_DEV - 1):
            o_r = lax.rem(me - h + N_DEV, N_DEV)
            o_l = lax.rem(me + h, N_DEV)
            rdma_r = pltpu.make_async_remote_copy(
                src_ref=w_r.at[o_r], dst_ref=w_r.at[o_r],
                send_sem=send_r.at[h], recv_sem=recv_r.at[h],
                device_id=(right,), device_id_type=pl.DeviceIdType.MESH,
            )
            rdma_l = pltpu.make_async_remote_copy(
                src_ref=w_l.at[o_l], dst_ref=w_l.at[o_l],
                send_sem=send_l.at[h], recv_sem=recv_l.at[h],
                device_id=(left,), device_id_type=pl.DeviceIdType.MESH,
            )
            rdma_r.start()
            rdma_l.start()
            consume(o_r, 0, w_r[o_r])
            consume(o_l, HALF, w_l[o_l])
            rdma_r.wait()
            rdma_l.wait()

        o_r_last = lax.rem(me + 1, N_DEV)
        o_l_last = lax.rem(me + N_DEV - 1, N_DEV)
        consume(o_r_last, 0, w_r[o_r_last])
        consume(o_l_last, HALF, w_l[o_l_last])

    return pl.pallas_call(
        body,
        out_shape=jax.ShapeDtypeStruct((n_tok, d_ff), jnp.float32),
        in_specs=[
            pl.BlockSpec(memory_space=pltpu.VMEM),
            pl.BlockSpec(memory_space=pltpu.VMEM),
            pl.BlockSpec(memory_space=pltpu.VMEM),
            pl.BlockSpec(memory_space=pltpu.VMEM),
        ],
        out_specs=pl.BlockSpec(memory_space=pltpu.VMEM),
        scratch_shapes=[
            pltpu.VMEM((N_DEV, HALF * 256, 512), jnp.bfloat16),
            pltpu.VMEM((N_DEV, HALF * 256, 512), jnp.bfloat16),
            pltpu.VMEM((N_DEV, 512, N_EXP_LOCAL), jnp.float32),
            pltpu.SemaphoreType.DMA((N_DEV - 1,)),
            pltpu.SemaphoreType.DMA((N_DEV - 1,)),
            pltpu.SemaphoreType.DMA((N_DEV - 1,)),
            pltpu.SemaphoreType.DMA((N_DEV - 1,)),
        ],
        compiler_params=pltpu.CompilerParams(
            collective_id=0,
            vmem_limit_bytes=100 * 1024 * 1024,
        ),
    )(x, router_W, route_idx, ew_bf)


# device time: 369085 ns/iter; 1.1870x vs baseline; 1.1209x over previous
import jax
import jax.numpy as jnp
from jax import lax
from jax.experimental import pallas as pl
from jax.experimental.pallas import tpu as pltpu

N_DEV = 32
N_EXP_LOCAL = 4
HALF = N_EXP_LOCAL // 2
N_SUB = 2
N_STEP = (N_DEV - 1) * N_SUB


def kernel(x, router_W, route_idx, expert_W):
    n_tok, d_model = x.shape
    d_ff = expert_W.shape[-1]
    n_exp = router_W.shape[-1]
    k_half = HALF * d_model

    ew_bf = expert_W.astype(jnp.bfloat16)

    def body(x_ref, rw_ref, idx_ref, ew_ref, out_ref,
             w_all, gate_s, send_sems, recv_sems):
        me = lax.axis_index("i")
        left = lax.rem(me + N_DEV - 1, N_DEV)
        right = lax.rem(me + 1, N_DEV)

        barrier_sem = pltpu.get_barrier_semaphore()
        for nbr in (left, right):
            pl.semaphore_signal(
                barrier_sem, inc=1,
                device_id=(nbr,), device_id_type=pl.DeviceIdType.MESH,
            )
        pl.semaphore_wait(barrier_sem, 2)

        w_all[N_SUB * me] = jnp.reshape(ew_ref[0:HALF], (k_half, d_ff))
        w_all[N_SUB * me + 1] = jnp.reshape(ew_ref[HALF:N_EXP_LOCAL], (k_half, d_ff))

        def fwd(g, k, do_start):
            o = lax.rem(me - g + N_DEV, N_DEV)
            slot = N_SUB * o + k
            r = pltpu.make_async_remote_copy(
                src_ref=w_all.at[slot], dst_ref=w_all.at[slot],
                send_sem=send_sems.at[N_SUB * g + k],
                recv_sem=recv_sems.at[N_SUB * g + k],
                device_id=(right,), device_id_type=pl.DeviceIdType.MESH,
            )
            if do_start:
                r.start()
            return r

        fwd(0, 0, True)
        fwd(0, 1, True)

        xf = x_ref[...]
        scores = jnp.dot(xf, rw_ref[...], preferred_element_type=jnp.float32)
        s_max = jnp.max(scores, axis=-1, keepdims=True)
        p = jnp.exp(scores - s_max)
        probs = p / jnp.sum(p, axis=-1, keepdims=True)
        ids = lax.broadcasted_iota(jnp.int32, (n_tok, n_exp), 1)
        oh0 = ids == idx_ref[:, 0:1]
        oh1 = ids == idx_ref[:, 1:2]
        g0 = jnp.sum(jnp.where(oh0, probs, 0.0), axis=-1, keepdims=True)
        g1 = jnp.sum(jnp.where(oh1, probs, 0.0), axis=-1, keepdims=True)
        gate = probs * (oh0 | oh1).astype(jnp.float32) / (g0 + g1)

        for o in range(N_DEV):
            gate_s[o] = gate[:, N_EXP_LOCAL * o : N_EXP_LOCAL * (o + 1)]

        xb = xf.astype(jnp.bfloat16)
        out_ref[...] = jnp.zeros((n_tok, d_ff), jnp.float32)

        def consume(o, k):
            g4 = gate_s[o]
            xg = jnp.concatenate(
                [
                    xb * g4[:, HALF * k + j : HALF * k + j + 1].astype(jnp.bfloat16)
                    for j in range(HALF)
                ],
                axis=1,
            )
            out_ref[...] = out_ref[...] + jnp.dot(
                xg, w_all[N_SUB * o + k], preferred_element_type=jnp.float32
            )

        consume(me, 0)
        consume(me, 1)

        for h in range(1, N_DEV):
            o = lax.rem(me - h + N_DEV, N_DEV)
            for k in range(N_SUB):
                fwd(h - 1, k, False).wait_recv()
                if h <= N_DEV - 2:
                    fwd(h, k, True)
                consume(o, k)

        for g in range(N_DEV - 1):
            for k in range(N_SUB):
                fwd(g, k, False).wait_send()

    return pl.pallas_call(
        body,
        out_shape=jax.ShapeDtypeStruct((n_tok, d_ff), jnp.float32),
        in_specs=[
            pl.BlockSpec(memory_space=pltpu.VMEM),
            pl.BlockSpec(memory_space=pltpu.VMEM),
            pl.BlockSpec(memory_space=pltpu.VMEM),
            pl.BlockSpec(memory_space=pltpu.VMEM),
        ],
        out_specs=pl.BlockSpec(memory_space=pltpu.VMEM),
        scratch_shapes=[
            pltpu.VMEM((N_DEV * N_SUB, HALF * 256, 512), jnp.bfloat16),
            pltpu.VMEM((N_DEV, 512, N_EXP_LOCAL), jnp.float32),
            pltpu.SemaphoreType.DMA((N_STEP,)),
            pltpu.SemaphoreType.DMA((N_STEP,)),
        ],
        compiler_params=pltpu.CompilerParams(
            collective_id=0,
            vmem_limit_bytes=100 * 1024 * 1024,
        ),
    )(x, router_W, route_idx, ew_bf)
